# baseline (speedup 1.0000x reference)
import functools

import jax
import jax.numpy as jnp
from jax import lax
from jax.experimental import pallas as pl
from jax.experimental.pallas import tpu as pltpu
from jax.experimental.pallas import tpu_sc as plsc

BT = 128
D_IN = 768
D_SAE = 24576
TOPK = 64
F_BLK = 2048
NB = D_SAE // F_BLK

NC, NS, L = 2, 16, 16
NW = NC * NS
ROWS_W = BT // NW
NV = D_SAE // L
CD = D_IN // L
NHIST = 256
CAP = 2048
NVC = CAP // L


def _tc_enc_body(x_ref, we_ref, be_ref, bd_ref, out_ref):
    xc = x_ref[...] - bd_ref[...]
    pre = lax.dot_general(xc, we_ref[...], (((1,), (0,)), ((), ())),
                          preferred_element_type=jnp.float32)
    out_ref[...] = jnp.maximum(pre + be_ref[...], 0.0)


def _tc_encode(x2d, w_enc, b_enc2d, b_dec2d):
    return pl.pallas_call(
        _tc_enc_body,
        grid=(NB,),
        in_specs=[
            pl.BlockSpec((BT, D_IN), lambda i: (0, 0)),
            pl.BlockSpec((D_IN, F_BLK), lambda i: (0, i)),
            pl.BlockSpec((1, F_BLK), lambda i: (0, i)),
            pl.BlockSpec((1, D_IN), lambda i: (0, 0)),
        ],
        out_specs=pl.BlockSpec((BT, F_BLK), lambda i: (0, i)),
        out_shape=jax.ShapeDtypeStruct((BT, D_SAE), jnp.float32),
    )(x2d, w_enc, b_enc2d, b_dec2d)


def _bcast_lane(vec, ln):
    return lax.gather(
        vec, jnp.full((L, 1), ln, jnp.int32),
        lax.GatherDimensionNumbers(
            offset_dims=(), collapsed_slice_dims=(0,), start_index_map=(0,)),
        (1,), mode=lax.GatherScatterMode.PROMISE_IN_BOUNDS)


def _sc_body(acts_hbm, wdec_hbm, bdec_hbm, sparse_hbm, out_hbm,
             row_a, row_b, srow_v, rows_v, hist_v, khi_v, kbits_v, oidx_v,
             cidx_v, cval_v, pidx_v, bvec_v, orow_v,
             sem_r, sem_o, sem_s, sem_g0, sem_g1, sem_g2, sem_g3):
    wid = lax.axis_index("s") * NC + lax.axis_index("c")
    zero16f = jnp.zeros((L,), jnp.float32)
    zero16i = jnp.zeros((L,), jnp.int32)
    ones16i = jnp.zeros((L,), jnp.int32) + 1
    neg16i = jnp.zeros((L,), jnp.int32) - 1
    lane = lax.iota(jnp.int32, L)
    lane256 = lane * NHIST
    sem_g = [sem_g0, sem_g1, sem_g2, sem_g3]
    NG = TOPK // L

    pltpu.sync_copy(bdec_hbm, bvec_v)
    def zs(i, c):
        srow_v[pl.ds(i * L, L)] = zero16f
        return c
    lax.fori_loop(0, NV, zs, jnp.int32(0))
    for q in range(NG):
        pidx_v[pl.ds(q * L, L)] = zero16i

    def zk(i, c):
        khi_v[pl.ds(i * L, L)] = zero16i
        return c
    lax.fori_loop(0, NVC, zk, jnp.int32(0))

    row0 = wid * ROWS_W
    cp_in = pltpu.async_copy(acts_hbm.at[row0], row_a, sem_r)
    cp_out = None
    cp_srow = None

    for r4 in range(ROWS_W):
        row = row0 + r4
        buf = row_a if r4 % 2 == 0 else row_b
        nxt = row_b if r4 % 2 == 0 else row_a
        cp_in.wait()
        if r4 + 1 < ROWS_W:
            cp_in = pltpu.async_copy(acts_hbm.at[row + 1], nxt, sem_r)

        def zh(i, c):
            for k in range(8):
                hist_v[pl.ds((i * 8 + k) * L, L)] = zero16i
            return c
        lax.fori_loop(0, (NHIST * L) // (8 * L), zh, jnp.int32(0))

        def s1(i, c):
            for k in range(8):
                v = buf[pl.ds((i * 8 + k) * L, L)]
                bits = plsc.bitcast(v, jnp.int32)
                e = lax.shift_right_logical(bits, 23)
                plsc.addupdate_scatter(hist_v, [lane256 + e], ones16i)
            return c
        lax.fori_loop(0, NV // 8, s1, jnp.int32(0))

        carry = zero16i
        run_e = neg16i
        for ec in range(NHIST // L - 1, -1, -1):
            tv = hist_v[pl.ds(ec * L, L)]
            for l2 in range(1, L):
                tv = tv + hist_v[pl.ds(l2 * NHIST + ec * L, L)]
            suf = lax.rev(plsc.cumsum(lax.rev(tv, (0,))), (0,)) + carry
            carry = _bcast_lane(suf, 0)
            run_e = jnp.maximum(run_e,
                                jnp.where(suf >= TOPK, lane + ec * L, -1))
        e_star = jnp.max(run_e)

        BPL = CAP // L
        lane_bpl = lane * BPL

        def s2(i, cnt):
            for k in range(8):
                v = buf[pl.ds((i * 8 + k) * L, L)]
                bits = plsc.bitcast(v, jnp.int32)
                e = lax.shift_right_logical(bits, 23)
                m = e >= e_star
                pos = lane_bpl + jnp.minimum(cnt, BPL - 1)
                plsc.store_scatter(khi_v, [pos], lane + (i * 8 + k) * L,
                                   mask=m)
                cnt = cnt + m.astype(jnp.int32)
            return cnt
        cntv = lax.fori_loop(0, NV // 8, s2, zero16i)
        mrows = jnp.minimum(jnp.max(cntv), BPL)
        m4 = (mrows + 3) // 4

        def mat(j, c):
            segi = plsc.load_gather(khi_v, [lane_bpl + j])
            valid = cntv > j
            vals = plsc.load_gather(buf, [segi])
            kb = jnp.where(valid, plsc.bitcast(vals, jnp.int32), -1)
            kbits_v[pl.ds(j * L, L)] = kb
            oidx_v[pl.ds(j * L, L)] = segi
            return c
        lax.fori_loop(0, mrows, mat, jnp.int32(0))

        def padk(j, c):
            kbits_v[pl.ds(j * L, L)] = neg16i
            return c
        lax.fori_loop(mrows, m4 * 4, padk, jnp.int32(0))

        def count_ge(c):
            def body(q, acc):
                for k in range(4):
                    kb = kbits_v[pl.ds((q * 4 + k) * L, L)]
                    acc = acc + (kb >= c).astype(jnp.int32)
                return acc
            return jnp.sum(lax.fori_loop(0, m4, body, zero16i))

        def bit_round(b, t):
            c = t | lax.shift_left(jnp.int32(1), 30 - b)
            return jnp.where(count_ge(c) >= TOPK, c, t)
        tbits = lax.fori_loop(0, 31, bit_round, jnp.int32(0))

        n_gt = count_ge(tbits + 1)
        n_need = TOPK - n_gt

        def count_eq_lt(c):
            def body(q, acc):
                for k in range(4):
                    kb = kbits_v[pl.ds((q * 4 + k) * L, L)]
                    oi = oidx_v[pl.ds((q * 4 + k) * L, L)]
                    acc = acc + ((kb == tbits) & (oi < c)).astype(jnp.int32)
                return acc
            return jnp.sum(lax.fori_loop(0, m4, body, zero16i))

        def tie_round(b, jc):
            c = jc | lax.shift_left(jnp.int32(1), 14 - b)
            return jnp.where(count_eq_lt(c) < n_need, c, jc)
        jcut = lax.fori_loop(0, 15, tie_round, jnp.int32(0))

        def choose(q, base):
            ms, css, pops, idv = [], [], [], []
            for k in range(4):
                kb = kbits_v[pl.ds((q * 4 + k) * L, L)]
                oi = oidx_v[pl.ds((q * 4 + k) * L, L)]
                m = (kb > tbits) | ((kb == tbits) & (oi <= jcut))
                ms.append(m)
                css.append(plsc.cumsum(m.astype(jnp.int32)))
                pops.append(plsc.all_reduce_population_count(m))
                idv.append(oi)
            bases = [base]
            for k in range(3):
                bases.append(bases[-1] + pops[k])
            for k in range(4):
                pos = jnp.minimum(bases[k] + css[k] - 1, TOPK - 1)
                plsc.store_scatter(cidx_v, [pos], idv[k], mask=ms[k])
            return bases[-1] + pops[3]
        lax.fori_loop(0, m4, choose, zero16i)

        for q in range(NG):
            idxs = cidx_v[pl.ds(q * L, L)]
            cval_v[pl.ds(q * L, L)] = plsc.load_gather(buf, [idxs])
        cps = [None] * NG
        for q in range(2):
            cps[q] = pltpu.async_copy(
                wdec_hbm.at[cidx_v.at[pl.ds(q * L, L)]],
                rows_v.at[pl.ds((q % 2) * L, L)], sem_g[q])

        if cp_srow is not None:
            cp_srow.wait()
        for q in range(NG):
            pi = pidx_v[pl.ds(q * L, L)]
            plsc.store_scatter(srow_v, [pi], zero16f)
        for q in range(NG):
            ci = cidx_v[pl.ds(q * L, L)]
            plsc.store_scatter(srow_v, [ci], cval_v[pl.ds(q * L, L)])
            pidx_v[pl.ds(q * L, L)] = ci
        cp_srow = pltpu.async_copy(srow_v, sparse_hbm.at[row], sem_s)

        if cp_out is not None:
            cp_out.wait()

        def init_c(c, carry):
            orow_v[pl.ds(c * L, L)] = bvec_v[pl.ds(c * L, L)]
            return carry
        lax.fori_loop(0, CD, init_c, jnp.int32(0))

        for q in range(NG):
            cps[q].wait()
            vals = cval_v[pl.ds(q * L, L)]

            def c_step(c, carry):
                acc0 = zero16f
                acc1 = zero16f
                for ln in range(L):
                    vb = _bcast_lane(vals, ln)
                    r = rows_v[(q % 2) * L + ln, pl.ds(c * L, L)]
                    if ln % 2 == 0:
                        acc0 = acc0 + vb * r
                    else:
                        acc1 = acc1 + vb * r
                orow_v[pl.ds(c * L, L)] += acc0 + acc1
                return carry
            lax.fori_loop(0, CD, c_step, jnp.int32(0))
            if q + 2 < NG:
                cps[q + 2] = pltpu.async_copy(
                    wdec_hbm.at[cidx_v.at[pl.ds((q + 2) * L, L)]],
                    rows_v.at[pl.ds(((q + 2) % 2) * L, L)], sem_g[q + 2])

        cp_out = pltpu.async_copy(orow_v, out_hbm.at[row], sem_o)

    cp_out.wait()
    cp_srow.wait()


def _sc_select_decode(acts, w_dec, b_dec):
    mesh = plsc.VectorSubcoreMesh(core_axis_name="c", subcore_axis_name="s")
    f = pl.kernel(
        _sc_body,
        mesh=mesh,
        compiler_params=pltpu.CompilerParams(needs_layout_passes=False),
        out_type=[
            jax.ShapeDtypeStruct((BT, D_SAE), jnp.float32),
            jax.ShapeDtypeStruct((BT, D_IN), jnp.float32),
        ],
        scratch_types=[
            pltpu.VMEM((D_SAE,), jnp.float32),
            pltpu.VMEM((D_SAE,), jnp.float32),
            pltpu.VMEM((D_SAE,), jnp.float32),
            pltpu.VMEM((2 * L, D_IN), jnp.float32),
            pltpu.VMEM((NHIST * L,), jnp.int32),
            pltpu.VMEM((CAP,), jnp.int32),
            pltpu.VMEM((CAP,), jnp.int32),
            pltpu.VMEM((CAP,), jnp.int32),
            pltpu.VMEM((TOPK,), jnp.int32),
            pltpu.VMEM((TOPK,), jnp.float32),
            pltpu.VMEM((TOPK,), jnp.int32),
            pltpu.VMEM((D_IN,), jnp.float32),
            pltpu.VMEM((D_IN,), jnp.float32),
            pltpu.SemaphoreType.DMA,
            pltpu.SemaphoreType.DMA,
            pltpu.SemaphoreType.DMA,
            pltpu.SemaphoreType.DMA,
            pltpu.SemaphoreType.DMA,
            pltpu.SemaphoreType.DMA,
            pltpu.SemaphoreType.DMA,
        ],
    )
    return f(acts, w_dec, b_dec)


@jax.jit
def _run(x2d, w_enc, b_enc2d, w_dec, b_dec2d):
    acts = _tc_encode(x2d, w_enc, b_enc2d, b_dec2d)
    sparse, recon = _sc_select_decode(acts, w_dec, b_dec2d.reshape(-1))
    return recon, sparse


def kernel(x, W_enc, b_enc, W_dec, b_dec):
    b, s, d_in = x.shape
    x2d = x.reshape(b * s, d_in)
    recon, sparse = _run(x2d, W_enc, b_enc.reshape(1, -1),
                         W_dec, b_dec.reshape(1, -1))
    return recon.reshape(b, s, d_in), sparse.reshape(b, s, -1)


# speedup vs baseline: 1.1770x; 1.1770x over previous
import functools

import jax
import jax.numpy as jnp
from jax import lax
from jax.experimental import pallas as pl
from jax.experimental.pallas import tpu as pltpu
from jax.experimental.pallas import tpu_sc as plsc

BT = 128
D_IN = 768
D_SAE = 24576
TOPK = 64
F_BLK = 2048
NB = D_SAE // F_BLK

NC, NS, L = 2, 16, 16
NW = NC * NS
ROWS_W = BT // NW
NV = D_SAE // L
CD = D_IN // L
NHIST = 256
CAP = 2048
NVC = CAP // L


def _tc_enc_body(x_ref, we_ref, be_ref, bd_ref, out_ref):
    xc = x_ref[...] - bd_ref[...]
    pre = lax.dot_general(xc, we_ref[...], (((1,), (0,)), ((), ())),
                          preferred_element_type=jnp.float32)
    out_ref[...] = jnp.maximum(pre + be_ref[...], 0.0)


def _tc_encode(x2d, w_enc, b_enc2d, b_dec2d):
    return pl.pallas_call(
        _tc_enc_body,
        grid=(NB,),
        in_specs=[
            pl.BlockSpec((BT, D_IN), lambda i: (0, 0)),
            pl.BlockSpec((D_IN, F_BLK), lambda i: (0, i)),
            pl.BlockSpec((1, F_BLK), lambda i: (0, i)),
            pl.BlockSpec((1, D_IN), lambda i: (0, 0)),
        ],
        out_specs=pl.BlockSpec((BT, F_BLK), lambda i: (0, i)),
        out_shape=jax.ShapeDtypeStruct((BT, D_SAE), jnp.float32),
    )(x2d, w_enc, b_enc2d, b_dec2d)


def _bcast_lane(vec, ln):
    return lax.gather(
        vec, jnp.full((L, 1), ln, jnp.int32),
        lax.GatherDimensionNumbers(
            offset_dims=(), collapsed_slice_dims=(0,), start_index_map=(0,)),
        (1,), mode=lax.GatherScatterMode.PROMISE_IN_BOUNDS)


def _sc_body(acts_hbm, wdec_hbm, bdec_hbm, sparse_hbm, out_hbm,
             row_a, row_b, srow_v, rows_v, hist_v, khi_v, kbits_v, oidx_v,
             cidx_v, cval_v, pidx_v, bvec_v, orow_v,
             sem_r, sem_o, sem_s, sem_g0, sem_g1, sem_g2, sem_g3):
    wid = lax.axis_index("s") * NC + lax.axis_index("c")
    zero16f = jnp.zeros((L,), jnp.float32)
    zero16i = jnp.zeros((L,), jnp.int32)
    ones16i = jnp.zeros((L,), jnp.int32) + 1
    neg16i = jnp.zeros((L,), jnp.int32) - 1
    lane = lax.iota(jnp.int32, L)
    lane256 = lane * NHIST
    sem_g = [sem_g0, sem_g1, sem_g2, sem_g3]
    NG = TOPK // L

    pltpu.sync_copy(bdec_hbm, bvec_v)
    def zs(i, c):
        srow_v[pl.ds(i * L, L)] = zero16f
        return c
    lax.fori_loop(0, NV, zs, jnp.int32(0))
    for q in range(NG):
        pidx_v[pl.ds(q * L, L)] = zero16i

    def zk(i, c):
        khi_v[pl.ds(i * L, L)] = zero16i
        return c
    lax.fori_loop(0, NVC, zk, jnp.int32(0))

    row0 = wid * ROWS_W
    cp_in = pltpu.async_copy(acts_hbm.at[row0], row_a, sem_r)
    cp_out = None
    cp_srow = None

    for r4 in range(ROWS_W):
        row = row0 + r4
        buf = row_a if r4 % 2 == 0 else row_b
        nxt = row_b if r4 % 2 == 0 else row_a
        cp_in.wait()
        if r4 + 1 < ROWS_W:
            cp_in = pltpu.async_copy(acts_hbm.at[row + 1], nxt, sem_r)

        def zh(i, c):
            for k in range(8):
                hist_v[pl.ds((i * 8 + k) * L, L)] = zero16i
            return c
        lax.fori_loop(0, (NHIST * L) // (8 * L), zh, jnp.int32(0))

        def s1(i, c):
            for k in range(8):
                v = buf[pl.ds((i * 8 + k) * L, L)]
                bits = plsc.bitcast(v, jnp.int32)
                e = lax.shift_right_logical(bits, 23)
                plsc.addupdate_scatter(hist_v, [lane256 + e], ones16i)
            return c
        lax.fori_loop(0, NV // 8, s1, jnp.int32(0))

        carry = zero16i
        run_e = neg16i
        for ec in range(NHIST // L - 1, -1, -1):
            tv = hist_v[pl.ds(ec * L, L)]
            for l2 in range(1, L):
                tv = tv + hist_v[pl.ds(l2 * NHIST + ec * L, L)]
            suf = lax.rev(plsc.cumsum(lax.rev(tv, (0,))), (0,)) + carry
            carry = _bcast_lane(suf, 0)
            run_e = jnp.maximum(run_e,
                                jnp.where(suf >= TOPK, lane + ec * L, -1))
        e_star = jnp.max(run_e)

        def s2(i, base):
            ms, css, pops = [], [], []
            for k in range(8):
                v = buf[pl.ds((i * 8 + k) * L, L)]
                bits = plsc.bitcast(v, jnp.int32)
                e = lax.shift_right_logical(bits, 23)
                m = e >= e_star
                ms.append(m)
                css.append(plsc.cumsum(m.astype(jnp.int32)))
                pops.append(plsc.all_reduce_population_count(m))
            bases = [base]
            for k in range(7):
                bases.append(bases[-1] + pops[k])
            for k in range(8):
                pos = jnp.minimum(bases[k] + css[k] - 1, CAP - 1)
                plsc.store_scatter(khi_v, [pos], lane + (i * 8 + k) * L,
                                   mask=ms[k])
            return bases[-1] + pops[7]
        basev = lax.fori_loop(0, NV // 8, s2, zero16i)
        ncand = jnp.minimum(jnp.max(basev), CAP)
        nvq = (ncand + (L - 1)) // L
        m4 = (ncand + (4 * L - 1)) // (4 * L)

        def zb(i, c):
            for k in range(8):
                kbits_v[pl.ds((i * 8 + k) * L, L)] = neg16i
            return c
        lax.fori_loop(0, NVC // 8, zb, jnp.int32(0))

        def mat(q, c):
            idxs = khi_v[pl.ds(q * L, L)]
            vals = plsc.load_gather(buf, [idxs])
            kbits_v[pl.ds(q * L, L)] = plsc.bitcast(vals, jnp.int32)
            return c
        lax.fori_loop(0, nvq, mat, jnp.int32(0))
        tail = (nvq - 1) * L
        plsc.store_scatter(kbits_v, [tail + lane], neg16i,
                           mask=(tail + lane) >= ncand)

        def count_ge(c):
            def body(q, acc):
                for k in range(4):
                    kb = kbits_v[pl.ds((q * 4 + k) * L, L)]
                    acc = acc + (kb >= c).astype(jnp.int32)
                return acc
            return jnp.sum(lax.fori_loop(0, m4, body, zero16i))

        def bit_round(b, t):
            c = t | lax.shift_left(jnp.int32(1), 22 - b)
            return jnp.where(count_ge(c) >= TOPK, c, t)
        tbits = lax.fori_loop(0, 23, bit_round,
                              lax.shift_left(e_star, 23))

        n_gt = count_ge(tbits + 1)
        n_need = TOPK - n_gt

        def count_eq_lt(c):
            def body(q, acc):
                for k in range(4):
                    kb = kbits_v[pl.ds((q * 4 + k) * L, L)]
                    arr = lane + (q * 4 + k) * L
                    acc = acc + ((kb == tbits) & (arr < c)).astype(jnp.int32)
                return acc
            return jnp.sum(lax.fori_loop(0, m4, body, zero16i))

        def tie_round(b, jc):
            c = jc | lax.shift_left(jnp.int32(1), 11 - b)
            return jnp.where(count_eq_lt(c) < n_need, c, jc)
        jcut = lax.fori_loop(0, 12, tie_round, jnp.int32(0))

        def choose(q, base):
            ms, css, pops, idv = [], [], [], []
            for k in range(4):
                kb = kbits_v[pl.ds((q * 4 + k) * L, L)]
                arr = lane + (q * 4 + k) * L
                m = (kb > tbits) | ((kb == tbits) & (arr <= jcut))
                ms.append(m)
                css.append(plsc.cumsum(m.astype(jnp.int32)))
                pops.append(plsc.all_reduce_population_count(m))
                idv.append(khi_v[pl.ds((q * 4 + k) * L, L)])
            bases = [base]
            for k in range(3):
                bases.append(bases[-1] + pops[k])
            for k in range(4):
                pos = jnp.minimum(bases[k] + css[k] - 1, TOPK - 1)
                plsc.store_scatter(cidx_v, [pos], idv[k], mask=ms[k])
            return bases[-1] + pops[3]
        lax.fori_loop(0, m4, choose, zero16i)

        for q in range(NG):
            idxs = cidx_v[pl.ds(q * L, L)]
            cval_v[pl.ds(q * L, L)] = plsc.load_gather(buf, [idxs])
        cps = [None] * NG
        for q in range(2):
            cps[q] = pltpu.async_copy(
                wdec_hbm.at[cidx_v.at[pl.ds(q * L, L)]],
                rows_v.at[pl.ds((q % 2) * L, L)], sem_g[q])

        if cp_srow is not None:
            cp_srow.wait()
        for q in range(NG):
            pi = pidx_v[pl.ds(q * L, L)]
            plsc.store_scatter(srow_v, [pi], zero16f)
        for q in range(NG):
            ci = cidx_v[pl.ds(q * L, L)]
            plsc.store_scatter(srow_v, [ci], cval_v[pl.ds(q * L, L)])
            pidx_v[pl.ds(q * L, L)] = ci
        cp_srow = pltpu.async_copy(srow_v, sparse_hbm.at[row], sem_s)

        if cp_out is not None:
            cp_out.wait()

        def init_c(c, carry):
            orow_v[pl.ds(c * L, L)] = bvec_v[pl.ds(c * L, L)]
            return carry
        lax.fori_loop(0, CD, init_c, jnp.int32(0))

        for q in range(NG):
            cps[q].wait()
            vals = cval_v[pl.ds(q * L, L)]

            def c_step(c, carry):
                acc0 = zero16f
                acc1 = zero16f
                for ln in range(L):
                    vb = _bcast_lane(vals, ln)
                    r = rows_v[(q % 2) * L + ln, pl.ds(c * L, L)]
                    if ln % 2 == 0:
                        acc0 = acc0 + vb * r
                    else:
                        acc1 = acc1 + vb * r
                orow_v[pl.ds(c * L, L)] += acc0 + acc1
                return carry
            lax.fori_loop(0, CD, c_step, jnp.int32(0))
            if q + 2 < NG:
                cps[q + 2] = pltpu.async_copy(
                    wdec_hbm.at[cidx_v.at[pl.ds((q + 2) * L, L)]],
                    rows_v.at[pl.ds(((q + 2) % 2) * L, L)], sem_g[q + 2])

        cp_out = pltpu.async_copy(orow_v, out_hbm.at[row], sem_o)

    cp_out.wait()
    cp_srow.wait()


def _sc_select_decode(acts, w_dec, b_dec):
    mesh = plsc.VectorSubcoreMesh(core_axis_name="c", subcore_axis_name="s")
    f = pl.kernel(
        _sc_body,
        mesh=mesh,
        compiler_params=pltpu.CompilerParams(needs_layout_passes=False),
        out_type=[
            jax.ShapeDtypeStruct((BT, D_SAE), jnp.float32),
            jax.ShapeDtypeStruct((BT, D_IN), jnp.float32),
        ],
        scratch_types=[
            pltpu.VMEM((D_SAE,), jnp.float32),
            pltpu.VMEM((D_SAE,), jnp.float32),
            pltpu.VMEM((D_SAE,), jnp.float32),
            pltpu.VMEM((2 * L, D_IN), jnp.float32),
            pltpu.VMEM((NHIST * L,), jnp.int32),
            pltpu.VMEM((CAP,), jnp.int32),
            pltpu.VMEM((CAP,), jnp.int32),
            pltpu.VMEM((CAP,), jnp.int32),
            pltpu.VMEM((TOPK,), jnp.int32),
            pltpu.VMEM((TOPK,), jnp.float32),
            pltpu.VMEM((TOPK,), jnp.int32),
            pltpu.VMEM((D_IN,), jnp.float32),
            pltpu.VMEM((D_IN,), jnp.float32),
            pltpu.SemaphoreType.DMA,
            pltpu.SemaphoreType.DMA,
            pltpu.SemaphoreType.DMA,
            pltpu.SemaphoreType.DMA,
            pltpu.SemaphoreType.DMA,
            pltpu.SemaphoreType.DMA,
            pltpu.SemaphoreType.DMA,
        ],
    )
    return f(acts, w_dec, b_dec)


@jax.jit
def _run(x2d, w_enc, b_enc2d, w_dec, b_dec2d):
    acts = _tc_encode(x2d, w_enc, b_enc2d, b_dec2d)
    sparse, recon = _sc_select_decode(acts, w_dec, b_dec2d.reshape(-1))
    return recon, sparse


def kernel(x, W_enc, b_enc, W_dec, b_dec):
    b, s, d_in = x.shape
    x2d = x.reshape(b * s, d_in)
    recon, sparse = _run(x2d, W_enc, b_enc.reshape(1, -1),
                         W_dec, b_dec.reshape(1, -1))
    return recon.reshape(b, s, d_in), sparse.reshape(b, s, -1)
